# Initial kernel scaffold; baseline (speedup 1.0000x reference)
import jax
import jax.numpy as jnp
from jax.experimental import pallas as pl


def kernel(node_emb, weight1, root1, bias1, weight2, root2, bias2, edge_index, edge_type):
    raise NotImplementedError("write your pallas kernel here")


# speedup vs baseline: 9.4612x; 9.4612x over previous
import functools

import jax
import jax.numpy as jnp
from jax import lax
from jax.experimental import pallas as pl
from jax.experimental.pallas import tpu as pltpu
from jax.experimental.pallas import tpu_sc as plsc

N = 50000
D = 80
R = 35
NB = 5
BLK = 16
E = 800000

NC = 2
NS = 16
NT = NC * NS

CHUNK = 128
NCHUNKS = E // CHUNK

HALF = 884736
DEG_TBL = 917504
DEG_TOT = 2 * HALF
DUMP_MASK = 8191

NHALF = 25000
ACC_ROWS = 25088
SC_DUMP = 25008
TE = 1000
TN = 1000

_mesh = plsc.VectorSubcoreMesh(core_axis_name="c", subcore_axis_name="s")


def _fill(ref, n, val, dtype):
    for i in range(n // 16):
        ref[pl.ds(i * 16, 16)] = jnp.full((16,), val, dtype)


@functools.partial(
    pl.kernel,
    mesh=_mesh,
    compiler_params=pltpu.CompilerParams(use_tc_tiling_on_sc=False),
    out_type=jax.ShapeDtypeStruct((DEG_TOT,), jnp.float32),
    scratch_types=[
        pltpu.VMEM((CHUNK,), jnp.int32),
        pltpu.VMEM((CHUNK,), jnp.int32),
        pltpu.VMEM((CHUNK,), jnp.int32),
        pltpu.VMEM((CHUNK,), jnp.float32),
        pltpu.VMEM((2048,), jnp.float32),
        pltpu.VMEM_SHARED((DEG_TBL,), jnp.float32),
    ],
)
def _deg_kernel(dst_hbm, ty_hbm, deg_hbm, dstv, tyv, idxv, onesv, zerov, tbl):
    c = lax.axis_index("c")
    s = lax.axis_index("s")
    _fill(onesv, CHUNK, 1.0, jnp.float32)
    _fill(zerov, 2048, 0.0, jnp.float32)
    zpt = DEG_TBL // NS
    for i in range(zpt // 2048):
        pltpu.sync_copy(zerov, tbl.at[pl.ds(s * zpt + i * 2048, 2048)])
    plsc.subcore_barrier()

    lane = lax.iota(jnp.int32, 16)
    half_lo = c * HALF

    def body(k, carry):
        ci = s + k * NS
        base = ci * CHUNK
        pltpu.sync_copy(dst_hbm.at[pl.ds(base, CHUNK)], dstv)
        pltpu.sync_copy(ty_hbm.at[pl.ds(base, CHUNK)], tyv)
        for j in range(CHUNK // 16):
            sl = pl.ds(j * 16, 16)
            idx = dstv[sl] * R + tyv[sl]
            loc = idx - half_lo
            oor = (loc < 0) | (loc >= HALF)
            dump = HALF + ((base + j * 16 + lane) & DUMP_MASK)
            idxv[sl] = jnp.where(oor, dump, loc)
        pltpu.sync_copy(onesv, tbl.at[idxv], add=True)
        return carry

    nck = 390 + (s < 10).astype(jnp.int32)
    lax.fori_loop(0, nck, body, 0)
    plsc.subcore_barrier()
    wpt = HALF // NS
    for i in range(wpt // 2048):
        off = s * wpt + i * 2048
        pltpu.sync_copy(tbl.at[pl.ds(off, 2048)], zerov)
        pltpu.sync_copy(zerov, deg_hbm.at[pl.ds(c * HALF + off, 2048)])


def _gather_body(with_w):
    def body(src_hbm, dst_hbm, ty_hbm, x_hbm, deg_hbm, *rest):
        if with_w:
            (g_hbm, w_hbm, srcv, dstv, tyv, widxv, degv, wv, rows, sem) = rest
        else:
            (g_hbm, srcv, rows, sem) = rest
        c = lax.axis_index("c")
        s = lax.axis_index("s")
        wid = s * NC + c

        def step(k, carry):
            base = (wid + k * NT) * CHUNK
            pltpu.sync_copy(src_hbm.at[pl.ds(base, CHUNK)], srcv)
            if with_w:
                pltpu.sync_copy(dst_hbm.at[pl.ds(base, CHUNK)], dstv)
                pltpu.sync_copy(ty_hbm.at[pl.ds(base, CHUNK)], tyv)
                for j in range(CHUNK // 16):
                    sl = pl.ds(j * 16, 16)
                    widxv[sl] = dstv[sl] * R + tyv[sl]
                pltpu.async_copy(deg_hbm.at[widxv], degv, sem).wait()
                for j in range(CHUNK // 16):
                    sl = pl.ds(j * 16, 16)
                    wv[sl] = 1.0 / degv[sl]
                pltpu.sync_copy(wv, w_hbm.at[pl.ds(base, CHUNK)])
            pltpu.async_copy(x_hbm.at[srcv], rows, sem).wait()
            pltpu.sync_copy(rows, g_hbm.at[pl.ds(base, CHUNK)])
            return carry

        nck = 195 + (wid < 10).astype(jnp.int32)
        lax.fori_loop(0, nck, step, 0)

    return body


_gatherw_kernel = functools.partial(
    pl.kernel,
    mesh=_mesh,
    compiler_params=pltpu.CompilerParams(use_tc_tiling_on_sc=False),
    out_type=(jax.ShapeDtypeStruct((E, D), jnp.float32),
              jax.ShapeDtypeStruct((E,), jnp.float32)),
    scratch_types=[
        pltpu.VMEM((CHUNK,), jnp.int32),
        pltpu.VMEM((CHUNK,), jnp.int32),
        pltpu.VMEM((CHUNK,), jnp.int32),
        pltpu.VMEM((CHUNK,), jnp.int32),
        pltpu.VMEM((CHUNK,), jnp.float32),
        pltpu.VMEM((CHUNK,), jnp.float32),
        pltpu.VMEM((CHUNK, D), jnp.float32),
        pltpu.SemaphoreType.DMA,
    ],
)(_gather_body(True))

_gather_kernel = functools.partial(
    pl.kernel,
    mesh=_mesh,
    compiler_params=pltpu.CompilerParams(use_tc_tiling_on_sc=False),
    out_type=jax.ShapeDtypeStruct((E, D), jnp.float32),
    scratch_types=[
        pltpu.VMEM((CHUNK,), jnp.int32),
        pltpu.VMEM((CHUNK, D), jnp.float32),
        pltpu.SemaphoreType.DMA,
    ],
)(_gather_body(False))


HD = D // 2


def _transform_body(ty_ref, w_ref, g_ref, wd_ref, ylo_ref, yhi_ref):
    g = g_ref[...]
    ty = ty_ref[...]
    acc = jnp.zeros((TE, D), jnp.float32)
    for r in range(R):
        m = jnp.where(ty == float(r), 1.0, 0.0)
        acc = acc + m * jnp.dot(g, wd_ref[r], preferred_element_type=jnp.float32)
    y = acc * w_ref[...]
    ylo_ref[...] = y[:, :HD]
    yhi_ref[...] = y[:, HD:]


def _transform(tycol, wcol, g, wd):
    return pl.pallas_call(
        _transform_body,
        grid=(E // TE,),
        in_specs=[
            pl.BlockSpec((TE, 1), lambda i: (i, 0)),
            pl.BlockSpec((TE, 1), lambda i: (i, 0)),
            pl.BlockSpec((TE, D), lambda i: (i, 0)),
            pl.BlockSpec((R, D, D), lambda i: (0, 0, 0)),
        ],
        out_specs=[pl.BlockSpec((TE, HD), lambda i: (i, 0)),
                   pl.BlockSpec((TE, HD), lambda i: (i, 0))],
        out_shape=[jax.ShapeDtypeStruct((E, HD), jnp.float32),
                   jax.ShapeDtypeStruct((E, HD), jnp.float32)],
    )(tycol, wcol, g, wd)


@functools.partial(
    pl.kernel,
    mesh=_mesh,
    compiler_params=pltpu.CompilerParams(use_tc_tiling_on_sc=False),
    out_type=(jax.ShapeDtypeStruct((2 * ACC_ROWS, HD), jnp.float32),
              jax.ShapeDtypeStruct((2 * ACC_ROWS, HD), jnp.float32)),
    scratch_types=[
        pltpu.VMEM((CHUNK,), jnp.int32),
        pltpu.VMEM((CHUNK,), jnp.int32),
        pltpu.VMEM((CHUNK, HD), jnp.float32),
        pltpu.VMEM((CHUNK, HD), jnp.float32),
        pltpu.VMEM_SHARED((ACC_ROWS, HD), jnp.float32),
    ],
)
def _scatter_kernel(dst_hbm, ylo_hbm, yhi_hbm, outlo_hbm, outhi_hbm,
                    dstv, idxv, rows, zrows, acc):
    c = lax.axis_index("c")
    s = lax.axis_index("s")
    for j in range(CHUNK):
        _fill(zrows.at[j], HD, 0.0, jnp.float32)
    lane = lax.iota(jnp.int32, 16)
    base_node = c * NHALF
    rpt = ACC_ROWS // NS

    for y_hbm, out_hbm in ((ylo_hbm, outlo_hbm), (yhi_hbm, outhi_hbm)):
        for i in range(rpt // CHUNK):
            pltpu.sync_copy(zrows, acc.at[pl.ds(s * rpt + i * CHUNK, CHUNK)])
        pltpu.sync_copy(
            zrows.at[pl.ds(0, rpt % CHUNK)],
            acc.at[pl.ds(s * rpt + (rpt // CHUNK) * CHUNK, rpt % CHUNK)])
        plsc.subcore_barrier()

        def step(k, carry):
            base = (s + k * NS) * CHUNK
            pltpu.sync_copy(dst_hbm.at[pl.ds(base, CHUNK)], dstv)
            pltpu.sync_copy(y_hbm.at[pl.ds(base, CHUNK)], rows)
            for j in range(CHUNK // 16):
                sl = pl.ds(j * 16, 16)
                loc = dstv[sl] - base_node
                oor = (loc < 0) | (loc >= NHALF)
                dump = SC_DUMP + ((j * 16 + lane) & 63)
                idxv[sl] = jnp.where(oor, dump, loc)
            pltpu.sync_copy(rows, acc.at[idxv], add=True)
            return carry

        nck = 390 + (s < 10).astype(jnp.int32)
        lax.fori_loop(0, nck, step, 0)
        plsc.subcore_barrier()
        for i in range(rpt // CHUNK):
            off = s * rpt + i * CHUNK
            pltpu.sync_copy(acc.at[pl.ds(off, CHUNK)], rows)
            pltpu.sync_copy(rows, out_hbm.at[pl.ds(c * ACC_ROWS + off, CHUNK)])
        rem = rpt % CHUNK
        off = s * rpt + (rpt // CHUNK) * CHUNK
        pltpu.sync_copy(acc.at[pl.ds(off, rem)], rows.at[pl.ds(0, rem)])
        pltpu.sync_copy(rows.at[pl.ds(0, rem)],
                        out_hbm.at[pl.ds(c * ACC_ROWS + off, rem)])
        plsc.subcore_barrier()


def _root_body(relu):
    def body(x_ref, o_ref, root_ref, b_ref, y_ref):
        y = o_ref[...] + jnp.dot(x_ref[...], root_ref[...],
                                 preferred_element_type=jnp.float32) + b_ref[...]
        if relu:
            y = jnp.maximum(y, 0.0)
        y_ref[...] = y
    return body


def _root_call(x, o, root, bias, relu):
    return pl.pallas_call(
        _root_body(relu),
        grid=(N // TN,),
        in_specs=[
            pl.BlockSpec((TN, D), lambda i: (i, 0)),
            pl.BlockSpec((TN, D), lambda i: (i, 0)),
            pl.BlockSpec((D, D), lambda i: (0, 0)),
            pl.BlockSpec((1, D), lambda i: (0, 0)),
        ],
        out_specs=pl.BlockSpec((TN, D), lambda i: (i, 0)),
        out_shape=jax.ShapeDtypeStruct((N, D), jnp.float32),
    )(x, o, root, bias)


def _block_diag_dense(w):
    out = jnp.zeros((R, NB, BLK, NB, BLK), w.dtype)
    for b in range(NB):
        out = out.at[:, b, :, b, :].set(w[:, b])
    return out.reshape(R, D, D)


def _unpad_out(outlo, outhi):
    lo = jnp.concatenate([outlo[:NHALF], outlo[ACC_ROWS:ACC_ROWS + NHALF]], 0)
    hi = jnp.concatenate([outhi[:NHALF], outhi[ACC_ROWS:ACC_ROWS + NHALF]], 0)
    return jnp.concatenate([lo, hi], axis=1)


def kernel(node_emb, weight1, root1, bias1, weight2, root2, bias2,
           edge_index, edge_type):
    src = edge_index[0].astype(jnp.int32)
    dst = edge_index[1].astype(jnp.int32)
    ty = edge_type.astype(jnp.int32)
    tycol = ty.astype(jnp.float32).reshape(E, 1)
    wd1 = _block_diag_dense(weight1)
    wd2 = _block_diag_dense(weight2)
    b1 = bias1.reshape(1, D)
    b2 = bias2.reshape(1, D)

    deg = _deg_kernel(dst, ty)
    g1, w = _gatherw_kernel(src, dst, ty, node_emb, deg)
    wcol = w.reshape(E, 1)

    y1lo, y1hi = _transform(tycol, wcol, g1, wd1)
    o1 = _unpad_out(*_scatter_kernel(dst, y1lo, y1hi))
    x1 = _root_call(node_emb, o1, root1, b1, relu=True)

    g2 = _gather_kernel(src, dst, ty, x1, deg)
    y2lo, y2hi = _transform(tycol, wcol, g2, wd2)
    o2 = _unpad_out(*_scatter_kernel(dst, y2lo, y2hi))
    x2 = _root_call(x1, o2, root2, b2, relu=False)
    return x2


# speedup vs baseline: 14.3335x; 1.5150x over previous
import functools

import jax
import jax.numpy as jnp
from jax import lax
from jax.experimental import pallas as pl
from jax.experimental.pallas import tpu as pltpu
from jax.experimental.pallas import tpu_sc as plsc

N = 50000
D = 80
R = 35
NB = 5
BLK = 16
E = 800000

NC = 2
NS = 16
NT = NC * NS

CHUNK = 128
NCHUNKS = E // CHUNK

HALF = 884736
DEG_TBL = 917504
DEG_TOT = 2 * HALF
DUMP_MASK = 8191

NHALF = 25000
ACC_ROWS = 25088
SC_DUMP = 25008
WLO = 48
WHI = 32
TN = 1000

_mesh = plsc.VectorSubcoreMesh(core_axis_name="c", subcore_axis_name="s")
_sc_params = pltpu.CompilerParams(use_tc_tiling_on_sc=False,
                                  needs_layout_passes=False)


def _fill(ref, n, val, dtype):
    for i in range(n // 16):
        ref[pl.ds(i * 16, 16)] = jnp.full((16,), val, dtype)


@functools.partial(
    pl.kernel,
    mesh=_mesh,
    compiler_params=_sc_params,
    out_type=jax.ShapeDtypeStruct((DEG_TOT,), jnp.float32),
    scratch_types=[
        pltpu.VMEM((CHUNK,), jnp.int32),
        pltpu.VMEM((CHUNK,), jnp.int32),
        pltpu.VMEM((CHUNK,), jnp.int32),
        pltpu.VMEM((CHUNK,), jnp.float32),
        pltpu.VMEM((2048,), jnp.float32),
        pltpu.VMEM_SHARED((DEG_TBL,), jnp.float32),
    ],
)
def _deg_kernel(dst_hbm, ty_hbm, deg_hbm, dstv, tyv, idxv, onesv, zerov, tbl):
    c = lax.axis_index("c")
    s = lax.axis_index("s")
    _fill(onesv, CHUNK, 1.0, jnp.float32)
    _fill(zerov, 2048, 0.0, jnp.float32)
    zpt = DEG_TBL // NS
    for i in range(zpt // 2048):
        pltpu.sync_copy(zerov, tbl.at[pl.ds(s * zpt + i * 2048, 2048)])
    plsc.subcore_barrier()

    lane = lax.iota(jnp.int32, 16)
    half_lo = c * HALF

    def body(k, carry):
        ci = s + k * NS
        base = ci * CHUNK
        pltpu.sync_copy(dst_hbm.at[pl.ds(base, CHUNK)], dstv)
        pltpu.sync_copy(ty_hbm.at[pl.ds(base, CHUNK)], tyv)
        for j in range(CHUNK // 16):
            sl = pl.ds(j * 16, 16)
            idx = dstv[sl] * R + tyv[sl]
            loc = idx - half_lo
            oor = (loc < 0) | (loc >= HALF)
            dump = HALF + ((base + j * 16 + lane) & DUMP_MASK)
            idxv[sl] = jnp.where(oor, dump, loc)
        pltpu.sync_copy(onesv, tbl.at[idxv], add=True)
        return carry

    nck = 390 + (s < 10).astype(jnp.int32)
    lax.fori_loop(0, nck, body, 0)
    plsc.subcore_barrier()
    wpt = HALF // NS
    for i in range(wpt // 2048):
        off = s * wpt + i * 2048
        pltpu.sync_copy(tbl.at[pl.ds(off, 2048)], zerov)
        pltpu.sync_copy(zerov, deg_hbm.at[pl.ds(c * HALF + off, 2048)])


def _gather_body(with_w):
    def body(*args):
        if with_w:
            (src_hbm, dst_hbm, ty_hbm, z_hbm, deg_hbm,
             ylo_hbm, yhi_hbm, w_hbm,
             srcv, dstv, tyv, zidxv, widxv, degv, wv, rows, rlo, rhi,
             sem) = args
        else:
            (src_hbm, ty_hbm, z_hbm, win_hbm,
             ylo_hbm, yhi_hbm,
             srcv, tyv, zidxv, wv, rows, rlo, rhi, sem) = args
        c = lax.axis_index("c")
        s = lax.axis_index("s")
        wid = s * NC + c

        def scale(j, carry):
            jv = jnp.full((16,), j, jnp.int32)
            wj = plsc.load_gather(wv, [jv])
            for t in range(WLO // 16):
                sl = pl.ds(t * 16, 16)
                rlo[j, sl] = rows[j, sl] * wj
            for t in range(WHI // 16):
                rhi[j, pl.ds(t * 16, 16)] = rows[j, pl.ds(WLO + t * 16, 16)] * wj
            return carry

        def step(k, carry):
            base = (wid + k * NT) * CHUNK
            pltpu.sync_copy(src_hbm.at[pl.ds(base, CHUNK)], srcv)
            pltpu.sync_copy(ty_hbm.at[pl.ds(base, CHUNK)], tyv)
            for j in range(CHUNK // 16):
                sl = pl.ds(j * 16, 16)
                zidxv[sl] = tyv[sl] * N + srcv[sl]
            if with_w:
                pltpu.sync_copy(dst_hbm.at[pl.ds(base, CHUNK)], dstv)
                for j in range(CHUNK // 16):
                    sl = pl.ds(j * 16, 16)
                    widxv[sl] = dstv[sl] * R + tyv[sl]
                pltpu.async_copy(deg_hbm.at[widxv], degv, sem).wait()
                for j in range(CHUNK // 16):
                    sl = pl.ds(j * 16, 16)
                    wv[sl] = 1.0 / degv[sl]
                pltpu.sync_copy(wv, w_hbm.at[pl.ds(base, CHUNK)])
            else:
                pltpu.sync_copy(win_hbm.at[pl.ds(base, CHUNK)], wv)
            pltpu.async_copy(z_hbm.at[zidxv], rows, sem).wait()
            lax.fori_loop(0, CHUNK, scale, 0)
            pltpu.sync_copy(rlo, ylo_hbm.at[pl.ds(base, CHUNK)])
            pltpu.sync_copy(rhi, yhi_hbm.at[pl.ds(base, CHUNK)])
            return carry

        nck = 195 + (wid < 10).astype(jnp.int32)
        lax.fori_loop(0, nck, step, 0)

    return body


_gatherw_kernel = functools.partial(
    pl.kernel,
    mesh=_mesh,
    compiler_params=_sc_params,
    out_type=(jax.ShapeDtypeStruct((E, WLO), jnp.float32),
              jax.ShapeDtypeStruct((E, WHI), jnp.float32),
              jax.ShapeDtypeStruct((E,), jnp.float32)),
    scratch_types=[
        pltpu.VMEM((CHUNK,), jnp.int32),
        pltpu.VMEM((CHUNK,), jnp.int32),
        pltpu.VMEM((CHUNK,), jnp.int32),
        pltpu.VMEM((CHUNK,), jnp.int32),
        pltpu.VMEM((CHUNK,), jnp.int32),
        pltpu.VMEM((CHUNK,), jnp.float32),
        pltpu.VMEM((CHUNK,), jnp.float32),
        pltpu.VMEM((CHUNK, D), jnp.float32),
        pltpu.VMEM((CHUNK, WLO), jnp.float32),
        pltpu.VMEM((CHUNK, WHI), jnp.float32),
        pltpu.SemaphoreType.DMA,
    ],
)(_gather_body(True))

_gather_kernel = functools.partial(
    pl.kernel,
    mesh=_mesh,
    compiler_params=_sc_params,
    out_type=(jax.ShapeDtypeStruct((E, WLO), jnp.float32),
              jax.ShapeDtypeStruct((E, WHI), jnp.float32)),
    scratch_types=[
        pltpu.VMEM((CHUNK,), jnp.int32),
        pltpu.VMEM((CHUNK,), jnp.int32),
        pltpu.VMEM((CHUNK,), jnp.int32),
        pltpu.VMEM((CHUNK,), jnp.float32),
        pltpu.VMEM((CHUNK, D), jnp.float32),
        pltpu.VMEM((CHUNK, WLO), jnp.float32),
        pltpu.VMEM((CHUNK, WHI), jnp.float32),
        pltpu.SemaphoreType.DMA,
    ],
)(_gather_body(False))


def _ztable_body(x_ref, wd_ref, z_ref):
    z_ref[...] = jnp.dot(x_ref[...], wd_ref[0],
                         preferred_element_type=jnp.float32)


def _ztable(x, wd):
    nt = N // TN
    return pl.pallas_call(
        _ztable_body,
        grid=(R, nt),
        in_specs=[
            pl.BlockSpec((TN, D), lambda r, i: (i, 0)),
            pl.BlockSpec((1, D, D), lambda r, i: (r, 0, 0)),
        ],
        out_specs=pl.BlockSpec((TN, D), lambda r, i: (r * nt + i, 0)),
        out_shape=jax.ShapeDtypeStruct((R * N, D), jnp.float32),
    )(x, wd)


def _make_scatter(width):
    @functools.partial(
        pl.kernel,
        mesh=_mesh,
        compiler_params=_sc_params,
        out_type=jax.ShapeDtypeStruct((2 * ACC_ROWS, width), jnp.float32),
        scratch_types=[
            pltpu.VMEM((CHUNK,), jnp.int32),
            pltpu.VMEM((CHUNK,), jnp.int32),
            pltpu.VMEM((CHUNK, width), jnp.float32),
            pltpu.VMEM((CHUNK, width), jnp.float32),
            pltpu.VMEM_SHARED((ACC_ROWS, width), jnp.float32),
        ],
    )
    def _scatter_kernel(dst_hbm, y_hbm, out_hbm, dstv, idxv, rows, zrows, acc):
        c = lax.axis_index("c")
        s = lax.axis_index("s")
        for j in range(CHUNK):
            _fill(zrows.at[j], width, 0.0, jnp.float32)
        rpt = ACC_ROWS // NS
        for i in range(rpt // CHUNK):
            pltpu.sync_copy(zrows, acc.at[pl.ds(s * rpt + i * CHUNK, CHUNK)])
        pltpu.sync_copy(
            zrows.at[pl.ds(0, rpt % CHUNK)],
            acc.at[pl.ds(s * rpt + (rpt // CHUNK) * CHUNK, rpt % CHUNK)])
        plsc.subcore_barrier()

        lane = lax.iota(jnp.int32, 16)
        base_node = c * NHALF

        def step(k, carry):
            base = (s + k * NS) * CHUNK
            pltpu.sync_copy(dst_hbm.at[pl.ds(base, CHUNK)], dstv)
            pltpu.sync_copy(y_hbm.at[pl.ds(base, CHUNK)], rows)
            for j in range(CHUNK // 16):
                sl = pl.ds(j * 16, 16)
                loc = dstv[sl] - base_node
                oor = (loc < 0) | (loc >= NHALF)
                dump = SC_DUMP + ((j * 16 + lane) & 63)
                idxv[sl] = jnp.where(oor, dump, loc)
            pltpu.sync_copy(rows, acc.at[idxv], add=True)
            return carry

        nck = 390 + (s < 10).astype(jnp.int32)
        lax.fori_loop(0, nck, step, 0)
        plsc.subcore_barrier()
        for i in range(rpt // CHUNK):
            off = s * rpt + i * CHUNK
            pltpu.sync_copy(acc.at[pl.ds(off, CHUNK)], rows)
            pltpu.sync_copy(rows, out_hbm.at[pl.ds(c * ACC_ROWS + off, CHUNK)])
        rem = rpt % CHUNK
        off = s * rpt + (rpt // CHUNK) * CHUNK
        pltpu.sync_copy(acc.at[pl.ds(off, rem)], rows.at[pl.ds(0, rem)])
        pltpu.sync_copy(rows.at[pl.ds(0, rem)],
                        out_hbm.at[pl.ds(c * ACC_ROWS + off, rem)])

    return _scatter_kernel


_scatter_lo = _make_scatter(WLO)
_scatter_hi = _make_scatter(WHI)


def _root_body(relu):
    def body(x_ref, olo_ref, ohi_ref, root_ref, b_ref, y_ref):
        o = jnp.concatenate([olo_ref[...], ohi_ref[...]], axis=1)
        y = o + jnp.dot(x_ref[...], root_ref[...],
                        preferred_element_type=jnp.float32) + b_ref[...]
        if relu:
            y = jnp.maximum(y, 0.0)
        y_ref[...] = y
    return body


def _root_call(x, olo, ohi, root, bias, relu):
    return pl.pallas_call(
        _root_body(relu),
        grid=(N // TN,),
        in_specs=[
            pl.BlockSpec((TN, D), lambda i: (i, 0)),
            pl.BlockSpec((TN, WLO), lambda i: (i, 0)),
            pl.BlockSpec((TN, WHI), lambda i: (i, 0)),
            pl.BlockSpec((D, D), lambda i: (0, 0)),
            pl.BlockSpec((1, D), lambda i: (0, 0)),
        ],
        out_specs=pl.BlockSpec((TN, D), lambda i: (i, 0)),
        out_shape=jax.ShapeDtypeStruct((N, D), jnp.float32),
    )(x, olo, ohi, root, bias)


def _block_diag_dense(w):
    out = jnp.zeros((R, NB, BLK, NB, BLK), w.dtype)
    for b in range(NB):
        out = out.at[:, b, :, b, :].set(w[:, b])
    return out.reshape(R, D, D)


def _unpad(outh):
    return jnp.concatenate(
        [outh[:NHALF], outh[ACC_ROWS:ACC_ROWS + NHALF]], axis=0)


def kernel(node_emb, weight1, root1, bias1, weight2, root2, bias2,
           edge_index, edge_type):
    src = edge_index[0].astype(jnp.int32)
    dst = edge_index[1].astype(jnp.int32)
    ty = edge_type.astype(jnp.int32)
    wd1 = _block_diag_dense(weight1)
    wd2 = _block_diag_dense(weight2)
    b1 = bias1.reshape(1, D)
    b2 = bias2.reshape(1, D)

    deg = _deg_kernel(dst, ty)

    z1 = _ztable(node_emb, wd1)
    y1lo, y1hi, w = _gatherw_kernel(src, dst, ty, z1, deg)
    o1lo = _unpad(_scatter_lo(dst, y1lo))
    o1hi = _unpad(_scatter_hi(dst, y1hi))
    x1 = _root_call(node_emb, o1lo, o1hi, root1, b1, relu=True)

    z2 = _ztable(x1, wd2)
    y2lo, y2hi = _gather_kernel(src, ty, z2, w)
    o2lo = _unpad(_scatter_lo(dst, y2lo))
    o2hi = _unpad(_scatter_hi(dst, y2hi))
    x2 = _root_call(x1, o2lo, o2hi, root2, b2, relu=False)
    return x2


# speedup vs baseline: 15.2027x; 1.0606x over previous
import functools

import jax
import jax.numpy as jnp
from jax import lax
from jax.experimental import pallas as pl
from jax.experimental.pallas import tpu as pltpu
from jax.experimental.pallas import tpu_sc as plsc

N = 50000
D = 80
R = 35
NB = 5
BLK = 16
E = 800000

NC = 2
NS = 16
NT = NC * NS

CHUNK = 128
NCHUNKS = E // CHUNK

HALF = 884736
DEG_TBL = 917504
DEG_TOT = 2 * HALF
DUMP_MASK = 8191

NHALF = 25000
ACC_ROWS = 25088
SC_DUMP = 25008
WLO = 48
WHI = 32
TN = 1000
TNZ = 2000

_mesh = plsc.VectorSubcoreMesh(core_axis_name="c", subcore_axis_name="s")
_sc_params = pltpu.CompilerParams(use_tc_tiling_on_sc=False,
                                  needs_layout_passes=False)


def _fill(ref, n, val, dtype):
    for i in range(n // 16):
        ref[pl.ds(i * 16, 16)] = jnp.full((16,), val, dtype)


@functools.partial(
    pl.kernel,
    mesh=_mesh,
    compiler_params=_sc_params,
    out_type=jax.ShapeDtypeStruct((DEG_TOT,), jnp.float32),
    scratch_types=[
        pltpu.VMEM((CHUNK,), jnp.int32),
        pltpu.VMEM((CHUNK,), jnp.int32),
        pltpu.VMEM((CHUNK,), jnp.int32),
        pltpu.VMEM((CHUNK,), jnp.float32),
        pltpu.VMEM((2048,), jnp.float32),
        pltpu.VMEM_SHARED((DEG_TBL,), jnp.float32),
    ],
)
def _deg_kernel(dst_hbm, ty_hbm, deg_hbm, dstv, tyv, idxv, onesv, zerov, tbl):
    c = lax.axis_index("c")
    s = lax.axis_index("s")
    _fill(onesv, CHUNK, 1.0, jnp.float32)
    _fill(zerov, 2048, 0.0, jnp.float32)
    zpt = DEG_TBL // NS
    for i in range(zpt // 2048):
        pltpu.sync_copy(zerov, tbl.at[pl.ds(s * zpt + i * 2048, 2048)])
    plsc.subcore_barrier()

    lane = lax.iota(jnp.int32, 16)
    half_lo = c * HALF

    def body(k, carry):
        ci = s + k * NS
        base = ci * CHUNK
        pltpu.sync_copy(dst_hbm.at[pl.ds(base, CHUNK)], dstv)
        pltpu.sync_copy(ty_hbm.at[pl.ds(base, CHUNK)], tyv)
        for j in range(CHUNK // 16):
            sl = pl.ds(j * 16, 16)
            idx = dstv[sl] * R + tyv[sl]
            loc = idx - half_lo
            oor = (loc < 0) | (loc >= HALF)
            dump = HALF + ((base + j * 16 + lane) & DUMP_MASK)
            idxv[sl] = jnp.where(oor, dump, loc)
        pltpu.sync_copy(onesv, tbl.at[idxv], add=True)
        return carry

    nck = 390 + (s < 10).astype(jnp.int32)
    lax.fori_loop(0, nck, body, 0)
    plsc.subcore_barrier()
    wpt = HALF // NS
    for i in range(wpt // 2048):
        off = s * wpt + i * 2048
        pltpu.sync_copy(tbl.at[pl.ds(off, 2048)], zerov)
        pltpu.sync_copy(zerov, deg_hbm.at[pl.ds(c * HALF + off, 2048)])


@functools.partial(
    pl.kernel,
    mesh=_mesh,
    compiler_params=_sc_params,
    out_type=jax.ShapeDtypeStruct((E,), jnp.float32),
    scratch_types=[
        pltpu.VMEM((CHUNK,), jnp.int32),
        pltpu.VMEM((CHUNK,), jnp.int32),
        pltpu.VMEM((CHUNK,), jnp.int32),
        pltpu.VMEM((CHUNK,), jnp.float32),
        pltpu.VMEM((CHUNK,), jnp.float32),
        pltpu.SemaphoreType.DMA,
    ],
)
def _w_kernel(dst_hbm, ty_hbm, deg_hbm, w_hbm, dstv, tyv, widxv, degv, wv, sem):
    c = lax.axis_index("c")
    s = lax.axis_index("s")
    wid = s * NC + c

    def step(k, carry):
        base = (wid + k * NT) * CHUNK
        pltpu.sync_copy(dst_hbm.at[pl.ds(base, CHUNK)], dstv)
        pltpu.sync_copy(ty_hbm.at[pl.ds(base, CHUNK)], tyv)
        for j in range(CHUNK // 16):
            sl = pl.ds(j * 16, 16)
            widxv[sl] = dstv[sl] * R + tyv[sl]
        pltpu.async_copy(deg_hbm.at[widxv], degv, sem).wait()
        for j in range(CHUNK // 16):
            sl = pl.ds(j * 16, 16)
            wv[sl] = 1.0 / degv[sl]
        pltpu.sync_copy(wv, w_hbm.at[pl.ds(base, CHUNK)])
        return carry

    nck = 195 + (wid < 10).astype(jnp.int32)
    lax.fori_loop(0, nck, step, 0)


@functools.partial(
    pl.kernel,
    mesh=_mesh,
    compiler_params=_sc_params,
    out_type=(jax.ShapeDtypeStruct((E, WLO), jnp.float32),
              jax.ShapeDtypeStruct((E, WHI), jnp.float32)),
    scratch_types=[
        pltpu.VMEM((CHUNK,), jnp.int32),
        pltpu.VMEM((CHUNK,), jnp.int32),
        pltpu.VMEM((CHUNK,), jnp.int32),
        pltpu.VMEM((CHUNK * 16,), jnp.float32),
        pltpu.VMEM((CHUNK, D), jnp.float32),
        pltpu.VMEM((CHUNK, WLO), jnp.float32),
        pltpu.VMEM((CHUNK, WHI), jnp.float32),
        pltpu.SemaphoreType.DMA,
    ],
)
def _gather_kernel(src_hbm, ty_hbm, z_hbm, w16_hbm, ylo_hbm, yhi_hbm,
                   srcv, tyv, zidxv, wrowv, rows, rlo, rhi, sem):
    c = lax.axis_index("c")
    s = lax.axis_index("s")
    wid = s * NC + c

    def scale(j, carry):
        wj = wrowv[pl.ds(j * 16, 16)]
        for t in range(WLO // 16):
            sl = pl.ds(t * 16, 16)
            rlo[j, sl] = rows[j, sl] * wj
        for t in range(WHI // 16):
            rhi[j, pl.ds(t * 16, 16)] = rows[j, pl.ds(WLO + t * 16, 16)] * wj
        return carry

    def step(k, carry):
        base = (wid + k * NT) * CHUNK
        pltpu.sync_copy(src_hbm.at[pl.ds(base, CHUNK)], srcv)
        pltpu.sync_copy(ty_hbm.at[pl.ds(base, CHUNK)], tyv)
        pltpu.sync_copy(w16_hbm.at[pl.ds(base * 16, CHUNK * 16)], wrowv)
        for j in range(CHUNK // 16):
            sl = pl.ds(j * 16, 16)
            zidxv[sl] = tyv[sl] * N + srcv[sl]
        pltpu.async_copy(z_hbm.at[zidxv], rows, sem).wait()
        lax.fori_loop(0, CHUNK, scale, 0)
        pltpu.sync_copy(rlo, ylo_hbm.at[pl.ds(base, CHUNK)])
        pltpu.sync_copy(rhi, yhi_hbm.at[pl.ds(base, CHUNK)])
        return carry

    nck = 195 + (wid < 10).astype(jnp.int32)
    lax.fori_loop(0, nck, step, 0)


def _ztable_body(x_ref, wd_ref, z_ref):
    z_ref[...] = jnp.dot(x_ref[...], wd_ref[0],
                         preferred_element_type=jnp.float32)


def _ztable(x, wd):
    nt = N // TNZ
    return pl.pallas_call(
        _ztable_body,
        grid=(nt, R),
        in_specs=[
            pl.BlockSpec((TNZ, D), lambda i, r: (i, 0)),
            pl.BlockSpec((1, D, D), lambda i, r: (r, 0, 0)),
        ],
        out_specs=pl.BlockSpec((TNZ, D), lambda i, r: (r * nt + i, 0)),
        out_shape=jax.ShapeDtypeStruct((R * N, D), jnp.float32),
    )(x, wd)


def _make_scatter(width):
    @functools.partial(
        pl.kernel,
        mesh=_mesh,
        compiler_params=_sc_params,
        out_type=jax.ShapeDtypeStruct((2 * ACC_ROWS, width), jnp.float32),
        scratch_types=[
            pltpu.VMEM((CHUNK,), jnp.int32),
            pltpu.VMEM((CHUNK,), jnp.int32),
            pltpu.VMEM((CHUNK, width), jnp.float32),
            pltpu.VMEM((CHUNK, width), jnp.float32),
            pltpu.VMEM_SHARED((ACC_ROWS, width), jnp.float32),
        ],
    )
    def _scatter_kernel(dst_hbm, y_hbm, out_hbm, dstv, idxv, rows, zrows, acc):
        c = lax.axis_index("c")
        s = lax.axis_index("s")
        for j in range(CHUNK):
            _fill(zrows.at[j], width, 0.0, jnp.float32)
        rpt = ACC_ROWS // NS
        for i in range(rpt // CHUNK):
            pltpu.sync_copy(zrows, acc.at[pl.ds(s * rpt + i * CHUNK, CHUNK)])
        pltpu.sync_copy(
            zrows.at[pl.ds(0, rpt % CHUNK)],
            acc.at[pl.ds(s * rpt + (rpt // CHUNK) * CHUNK, rpt % CHUNK)])
        plsc.subcore_barrier()

        lane = lax.iota(jnp.int32, 16)
        base_node = c * NHALF

        def step(k, carry):
            base = (s + k * NS) * CHUNK
            pltpu.sync_copy(dst_hbm.at[pl.ds(base, CHUNK)], dstv)
            pltpu.sync_copy(y_hbm.at[pl.ds(base, CHUNK)], rows)
            for j in range(CHUNK // 16):
                sl = pl.ds(j * 16, 16)
                loc = dstv[sl] - base_node
                oor = (loc < 0) | (loc >= NHALF)
                dump = SC_DUMP + ((j * 16 + lane) & 63)
                idxv[sl] = jnp.where(oor, dump, loc)
            pltpu.sync_copy(rows, acc.at[idxv], add=True)
            return carry

        nck = 390 + (s < 10).astype(jnp.int32)
        lax.fori_loop(0, nck, step, 0)
        plsc.subcore_barrier()
        for i in range(rpt // CHUNK):
            off = s * rpt + i * CHUNK
            pltpu.sync_copy(acc.at[pl.ds(off, CHUNK)], rows)
            pltpu.sync_copy(rows, out_hbm.at[pl.ds(c * ACC_ROWS + off, CHUNK)])
        rem = rpt % CHUNK
        off = s * rpt + (rpt // CHUNK) * CHUNK
        pltpu.sync_copy(acc.at[pl.ds(off, rem)], rows.at[pl.ds(0, rem)])
        pltpu.sync_copy(rows.at[pl.ds(0, rem)],
                        out_hbm.at[pl.ds(c * ACC_ROWS + off, rem)])

    return _scatter_kernel


_scatter_lo = _make_scatter(WLO)
_scatter_hi = _make_scatter(WHI)


def _root_body(relu):
    def body(x_ref, olo_ref, ohi_ref, root_ref, b_ref, y_ref):
        o = jnp.concatenate([olo_ref[...], ohi_ref[...]], axis=1)
        y = o + jnp.dot(x_ref[...], root_ref[...],
                        preferred_element_type=jnp.float32) + b_ref[...]
        if relu:
            y = jnp.maximum(y, 0.0)
        y_ref[...] = y
    return body


def _root_call(x, olo, ohi, root, bias, relu):
    return pl.pallas_call(
        _root_body(relu),
        grid=(N // TN,),
        in_specs=[
            pl.BlockSpec((TN, D), lambda i: (i, 0)),
            pl.BlockSpec((TN, WLO), lambda i: (i, 0)),
            pl.BlockSpec((TN, WHI), lambda i: (i, 0)),
            pl.BlockSpec((D, D), lambda i: (0, 0)),
            pl.BlockSpec((1, D), lambda i: (0, 0)),
        ],
        out_specs=pl.BlockSpec((TN, D), lambda i: (i, 0)),
        out_shape=jax.ShapeDtypeStruct((N, D), jnp.float32),
    )(x, olo, ohi, root, bias)


def _block_diag_dense(w):
    out = jnp.zeros((R, NB, BLK, NB, BLK), w.dtype)
    for b in range(NB):
        out = out.at[:, b, :, b, :].set(w[:, b])
    return out.reshape(R, D, D)


def _unpad(outh):
    return jnp.concatenate(
        [outh[:NHALF], outh[ACC_ROWS:ACC_ROWS + NHALF]], axis=0)


def kernel(node_emb, weight1, root1, bias1, weight2, root2, bias2,
           edge_index, edge_type):
    src = edge_index[0].astype(jnp.int32)
    dst = edge_index[1].astype(jnp.int32)
    ty = edge_type.astype(jnp.int32)
    wd1 = _block_diag_dense(weight1)
    wd2 = _block_diag_dense(weight2)
    b1 = bias1.reshape(1, D)
    b2 = bias2.reshape(1, D)

    deg = _deg_kernel(dst, ty)
    w = _w_kernel(dst, ty, deg)
    w16 = jnp.broadcast_to(w[:, None], (E, 16)).reshape(E * 16)

    z1 = _ztable(node_emb, wd1)
    y1lo, y1hi = _gather_kernel(src, ty, z1, w16)
    o1lo = _unpad(_scatter_lo(dst, y1lo))
    o1hi = _unpad(_scatter_hi(dst, y1hi))
    x1 = _root_call(node_emb, o1lo, o1hi, root1, b1, relu=True)

    z2 = _ztable(x1, wd2)
    y2lo, y2hi = _gather_kernel(src, ty, z2, w16)
    o2lo = _unpad(_scatter_lo(dst, y2lo))
    o2hi = _unpad(_scatter_hi(dst, y2hi))
    x2 = _root_call(x1, o2lo, o2hi, root2, b2, relu=False)
    return x2


# speedup vs baseline: 16.7220x; 1.0999x over previous
import functools

import jax
import jax.numpy as jnp
from jax import lax
from jax.experimental import pallas as pl
from jax.experimental.pallas import tpu as pltpu
from jax.experimental.pallas import tpu_sc as plsc

N = 50000
D = 80
R = 35
NB = 5
BLK = 16
E = 800000

NC = 2
NS = 16
NT = NC * NS

CHUNK = 128
NCHUNKS = E // CHUNK

HALF = 884736
DEG_TBL = 917504
DEG_TOT = 2 * HALF
DUMP_MASK = 8191

NHALF = 25000
ACC_ROWS = 25088
SC_DUMP = 25008
WLO = 48
WHI = 32
TN = 1000
TNZ = 2000

_mesh = plsc.VectorSubcoreMesh(core_axis_name="c", subcore_axis_name="s")
_sc_params = pltpu.CompilerParams(use_tc_tiling_on_sc=False,
                                  needs_layout_passes=False)


def _fill(ref, n, val, dtype):
    for i in range(n // 16):
        ref[pl.ds(i * 16, 16)] = jnp.full((16,), val, dtype)


@functools.partial(
    pl.kernel,
    mesh=_mesh,
    compiler_params=_sc_params,
    out_type=jax.ShapeDtypeStruct((DEG_TOT,), jnp.float32),
    scratch_types=[
        pltpu.VMEM((CHUNK,), jnp.int32),
        pltpu.VMEM((CHUNK,), jnp.int32),
        pltpu.VMEM((CHUNK,), jnp.int32),
        pltpu.VMEM((CHUNK,), jnp.float32),
        pltpu.VMEM((2048,), jnp.float32),
        pltpu.VMEM_SHARED((DEG_TBL,), jnp.float32),
    ],
)
def _deg_kernel(dst_hbm, ty_hbm, deg_hbm, dstv, tyv, idxv, onesv, zerov, tbl):
    c = lax.axis_index("c")
    s = lax.axis_index("s")
    _fill(onesv, CHUNK, 1.0, jnp.float32)
    _fill(zerov, 2048, 0.0, jnp.float32)
    zpt = DEG_TBL // NS
    for i in range(zpt // 2048):
        pltpu.sync_copy(zerov, tbl.at[pl.ds(s * zpt + i * 2048, 2048)])
    plsc.subcore_barrier()

    lane = lax.iota(jnp.int32, 16)
    half_lo = c * HALF

    def body(k, carry):
        ci = s + k * NS
        base = ci * CHUNK
        pltpu.sync_copy(dst_hbm.at[pl.ds(base, CHUNK)], dstv)
        pltpu.sync_copy(ty_hbm.at[pl.ds(base, CHUNK)], tyv)
        for j in range(CHUNK // 16):
            sl = pl.ds(j * 16, 16)
            idx = dstv[sl] * R + tyv[sl]
            loc = idx - half_lo
            oor = (loc < 0) | (loc >= HALF)
            dump = HALF + ((base + j * 16 + lane) & DUMP_MASK)
            idxv[sl] = jnp.where(oor, dump, loc)
        pltpu.sync_copy(onesv, tbl.at[idxv], add=True)
        return carry

    nck = 390 + (s < 10).astype(jnp.int32)
    lax.fori_loop(0, nck, body, 0)
    plsc.subcore_barrier()
    wpt = HALF // NS
    for i in range(wpt // 2048):
        off = s * wpt + i * 2048
        pltpu.sync_copy(tbl.at[pl.ds(off, 2048)], zerov)
        pltpu.sync_copy(zerov, deg_hbm.at[pl.ds(c * HALF + off, 2048)])


def _gather_body(with_w):
    def body(*args):
        if with_w:
            (src_hbm, dst_hbm, ty_hbm, z_hbm, deg_hbm,
             ylo_hbm, yhi_hbm, w_hbm,
             srcv, dstv, tyv, zidxv, widxv, degv, wv, rows, rlo, rhi,
             sem) = args
        else:
            (src_hbm, ty_hbm, z_hbm, win_hbm,
             ylo_hbm, yhi_hbm,
             srcv, tyv, zidxv, wv, rows, rlo, rhi, sem) = args
        c = lax.axis_index("c")
        s = lax.axis_index("s")
        wid = s * NC + c

        def scale(j, carry):
            jv = jnp.full((16,), j, jnp.int32)
            wj = plsc.load_gather(wv, [jv])
            for t in range(WLO // 16):
                sl = pl.ds(t * 16, 16)
                rlo[j, sl] = rows[j, sl] * wj
            for t in range(WHI // 16):
                rhi[j, pl.ds(t * 16, 16)] = rows[j, pl.ds(WLO + t * 16, 16)] * wj
            return carry

        def step(k, carry):
            base = (wid + k * NT) * CHUNK
            pltpu.sync_copy(src_hbm.at[pl.ds(base, CHUNK)], srcv)
            pltpu.sync_copy(ty_hbm.at[pl.ds(base, CHUNK)], tyv)
            for j in range(CHUNK // 16):
                sl = pl.ds(j * 16, 16)
                zidxv[sl] = tyv[sl] * N + srcv[sl]
            if with_w:
                pltpu.sync_copy(dst_hbm.at[pl.ds(base, CHUNK)], dstv)
                for j in range(CHUNK // 16):
                    sl = pl.ds(j * 16, 16)
                    widxv[sl] = dstv[sl] * R + tyv[sl]
                pltpu.async_copy(deg_hbm.at[widxv], degv, sem).wait()
                for j in range(CHUNK // 16):
                    sl = pl.ds(j * 16, 16)
                    wv[sl] = 1.0 / degv[sl]
                pltpu.sync_copy(wv, w_hbm.at[pl.ds(base, CHUNK)])
            else:
                pltpu.sync_copy(win_hbm.at[pl.ds(base, CHUNK)], wv)
            pltpu.async_copy(z_hbm.at[zidxv], rows, sem).wait()
            lax.fori_loop(0, CHUNK, scale, 0)
            pltpu.sync_copy(rlo, ylo_hbm.at[pl.ds(base, CHUNK)])
            pltpu.sync_copy(rhi, yhi_hbm.at[pl.ds(base, CHUNK)])
            return carry

        nck = 195 + (wid < 10).astype(jnp.int32)
        lax.fori_loop(0, nck, step, 0)

    return body


_gatherw_kernel = functools.partial(
    pl.kernel,
    mesh=_mesh,
    compiler_params=_sc_params,
    out_type=(jax.ShapeDtypeStruct((E, WLO), jnp.float32),
              jax.ShapeDtypeStruct((E, WHI), jnp.float32),
              jax.ShapeDtypeStruct((E,), jnp.float32)),
    scratch_types=[
        pltpu.VMEM((CHUNK,), jnp.int32),
        pltpu.VMEM((CHUNK,), jnp.int32),
        pltpu.VMEM((CHUNK,), jnp.int32),
        pltpu.VMEM((CHUNK,), jnp.int32),
        pltpu.VMEM((CHUNK,), jnp.int32),
        pltpu.VMEM((CHUNK,), jnp.float32),
        pltpu.VMEM((CHUNK,), jnp.float32),
        pltpu.VMEM((CHUNK, D), jnp.float32),
        pltpu.VMEM((CHUNK, WLO), jnp.float32),
        pltpu.VMEM((CHUNK, WHI), jnp.float32),
        pltpu.SemaphoreType.DMA,
    ],
)(_gather_body(True))

_gather_kernel = functools.partial(
    pl.kernel,
    mesh=_mesh,
    compiler_params=_sc_params,
    out_type=(jax.ShapeDtypeStruct((E, WLO), jnp.float32),
              jax.ShapeDtypeStruct((E, WHI), jnp.float32)),
    scratch_types=[
        pltpu.VMEM((CHUNK,), jnp.int32),
        pltpu.VMEM((CHUNK,), jnp.int32),
        pltpu.VMEM((CHUNK,), jnp.int32),
        pltpu.VMEM((CHUNK,), jnp.float32),
        pltpu.VMEM((CHUNK, D), jnp.float32),
        pltpu.VMEM((CHUNK, WLO), jnp.float32),
        pltpu.VMEM((CHUNK, WHI), jnp.float32),
        pltpu.SemaphoreType.DMA,
    ],
)(_gather_body(False))


def _ztable_body(x_ref, wd_ref, z_ref):
    z_ref[...] = jnp.dot(x_ref[...], wd_ref[0],
                         preferred_element_type=jnp.float32)


def _ztable(x, wd):
    nt = N // TNZ
    return pl.pallas_call(
        _ztable_body,
        grid=(nt, R),
        in_specs=[
            pl.BlockSpec((TNZ, D), lambda i, r: (i, 0)),
            pl.BlockSpec((1, D, D), lambda i, r: (r, 0, 0)),
        ],
        out_specs=pl.BlockSpec((TNZ, D), lambda i, r: (r * nt + i, 0)),
        out_shape=jax.ShapeDtypeStruct((R * N, D), jnp.float32),
    )(x, wd)


def _make_scatter(width):
    @functools.partial(
        pl.kernel,
        mesh=_mesh,
        compiler_params=_sc_params,
        out_type=jax.ShapeDtypeStruct((2 * ACC_ROWS, width), jnp.float32),
        scratch_types=[
            pltpu.VMEM((CHUNK,), jnp.int32),
            pltpu.VMEM((CHUNK,), jnp.int32),
            pltpu.VMEM((CHUNK, width), jnp.float32),
            pltpu.VMEM((CHUNK, width), jnp.float32),
            pltpu.VMEM_SHARED((ACC_ROWS, width), jnp.float32),
        ],
    )
    def _scatter_kernel(dst_hbm, y_hbm, out_hbm, dstv, idxv, rows, zrows, acc):
        c = lax.axis_index("c")
        s = lax.axis_index("s")
        for j in range(CHUNK):
            _fill(zrows.at[j], width, 0.0, jnp.float32)
        rpt = ACC_ROWS // NS
        for i in range(rpt // CHUNK):
            pltpu.sync_copy(zrows, acc.at[pl.ds(s * rpt + i * CHUNK, CHUNK)])
        pltpu.sync_copy(
            zrows.at[pl.ds(0, rpt % CHUNK)],
            acc.at[pl.ds(s * rpt + (rpt // CHUNK) * CHUNK, rpt % CHUNK)])
        plsc.subcore_barrier()

        lane = lax.iota(jnp.int32, 16)
        base_node = c * NHALF

        def step(k, carry):
            base = (s + k * NS) * CHUNK
            pltpu.sync_copy(dst_hbm.at[pl.ds(base, CHUNK)], dstv)
            pltpu.sync_copy(y_hbm.at[pl.ds(base, CHUNK)], rows)
            for j in range(CHUNK // 16):
                sl = pl.ds(j * 16, 16)
                loc = dstv[sl] - base_node
                oor = (loc < 0) | (loc >= NHALF)
                dump = SC_DUMP + ((j * 16 + lane) & 63)
                idxv[sl] = jnp.where(oor, dump, loc)
            pltpu.sync_copy(rows, acc.at[idxv], add=True)
            return carry

        nck = 390 + (s < 10).astype(jnp.int32)
        lax.fori_loop(0, nck, step, 0)
        plsc.subcore_barrier()
        for i in range(rpt // CHUNK):
            off = s * rpt + i * CHUNK
            pltpu.sync_copy(acc.at[pl.ds(off, CHUNK)], rows)
            pltpu.sync_copy(rows, out_hbm.at[pl.ds(c * ACC_ROWS + off, CHUNK)])
        rem = rpt % CHUNK
        off = s * rpt + (rpt // CHUNK) * CHUNK
        pltpu.sync_copy(acc.at[pl.ds(off, rem)], rows.at[pl.ds(0, rem)])
        pltpu.sync_copy(rows.at[pl.ds(0, rem)],
                        out_hbm.at[pl.ds(c * ACC_ROWS + off, rem)])

    return _scatter_kernel


_scatter_lo = _make_scatter(WLO)
_scatter_hi = _make_scatter(WHI)


def _root_body(relu):
    def body(x_ref, olo_ref, ohi_ref, root_ref, b_ref, y_ref):
        o = jnp.concatenate([olo_ref[...], ohi_ref[...]], axis=1)
        y = o + jnp.dot(x_ref[...], root_ref[...],
                        preferred_element_type=jnp.float32) + b_ref[...]
        if relu:
            y = jnp.maximum(y, 0.0)
        y_ref[...] = y
    return body


def _root_call(x, olo, ohi, root, bias, relu):
    return pl.pallas_call(
        _root_body(relu),
        grid=(N // TN,),
        in_specs=[
            pl.BlockSpec((TN, D), lambda i: (i, 0)),
            pl.BlockSpec((TN, WLO), lambda i: (i, 0)),
            pl.BlockSpec((TN, WHI), lambda i: (i, 0)),
            pl.BlockSpec((D, D), lambda i: (0, 0)),
            pl.BlockSpec((1, D), lambda i: (0, 0)),
        ],
        out_specs=pl.BlockSpec((TN, D), lambda i: (i, 0)),
        out_shape=jax.ShapeDtypeStruct((N, D), jnp.float32),
    )(x, olo, ohi, root, bias)


def _block_diag_dense(w):
    out = jnp.zeros((R, NB, BLK, NB, BLK), w.dtype)
    for b in range(NB):
        out = out.at[:, b, :, b, :].set(w[:, b])
    return out.reshape(R, D, D)


def _unpad(outh):
    return jnp.concatenate(
        [outh[:NHALF], outh[ACC_ROWS:ACC_ROWS + NHALF]], axis=0)


def kernel(node_emb, weight1, root1, bias1, weight2, root2, bias2,
           edge_index, edge_type):
    src = edge_index[0].astype(jnp.int32)
    dst = edge_index[1].astype(jnp.int32)
    ty = edge_type.astype(jnp.int32)
    wd1 = _block_diag_dense(weight1)
    wd2 = _block_diag_dense(weight2)
    b1 = bias1.reshape(1, D)
    b2 = bias2.reshape(1, D)

    deg = _deg_kernel(dst, ty)

    z1 = _ztable(node_emb, wd1)
    y1lo, y1hi, w = _gatherw_kernel(src, dst, ty, z1, deg)
    o1lo = _unpad(_scatter_lo(dst, y1lo))
    o1hi = _unpad(_scatter_hi(dst, y1hi))
    x1 = _root_call(node_emb, o1lo, o1hi, root1, b1, relu=True)

    z2 = _ztable(x1, wd2)
    y2lo, y2hi = _gather_kernel(src, ty, z2, w)
    o2lo = _unpad(_scatter_lo(dst, y2lo))
    o2hi = _unpad(_scatter_hi(dst, y2hi))
    x2 = _root_call(x1, o2lo, o2hi, root2, b2, relu=False)
    return x2


# speedup vs baseline: 19.4103x; 1.1608x over previous
import functools

import jax
import jax.numpy as jnp
from jax import lax
from jax.experimental import pallas as pl
from jax.experimental.pallas import tpu as pltpu
from jax.experimental.pallas import tpu_sc as plsc

N = 50000
D = 80
R = 35
NB = 5
BLK = 16
E = 800000

NC = 2
NS = 16
NT = NC * NS

CHUNK = 128
NCHUNKS = E // CHUNK

HALF = 884736
DEG_TBL = 917504
DEG_TOT = 2 * HALF
DUMP_MASK = 8191

NHALF = 25000
ACC_ROWS = 25088
SC_DUMP = 25008
WLO = 48
WHI = 32
TN = 1000
TNZ = 2000

_mesh = plsc.VectorSubcoreMesh(core_axis_name="c", subcore_axis_name="s")
_sc_params = pltpu.CompilerParams(use_tc_tiling_on_sc=False,
                                  needs_layout_passes=False)


def _fill(ref, n, val, dtype):
    for i in range(n // 16):
        ref[pl.ds(i * 16, 16)] = jnp.full((16,), val, dtype)


@functools.partial(
    pl.kernel,
    mesh=_mesh,
    compiler_params=_sc_params,
    out_type=jax.ShapeDtypeStruct((DEG_TOT,), jnp.float32),
    scratch_types=[
        pltpu.VMEM((CHUNK,), jnp.int32),
        pltpu.VMEM((CHUNK,), jnp.int32),
        pltpu.VMEM((CHUNK,), jnp.int32),
        pltpu.VMEM((CHUNK,), jnp.float32),
        pltpu.VMEM((2048,), jnp.float32),
        pltpu.VMEM_SHARED((DEG_TBL,), jnp.float32),
        pltpu.SemaphoreType.DMA,
        pltpu.SemaphoreType.DMA,
    ],
)
def _deg_kernel(dst_hbm, ty_hbm, deg_hbm, dstv, tyv, idxv, onesv, zerov, tbl,
                sem, sem2):
    c = lax.axis_index("c")
    s = lax.axis_index("s")
    _fill(onesv, CHUNK, 1.0, jnp.float32)
    _fill(zerov, 2048, 0.0, jnp.float32)
    zpt = DEG_TBL // NS
    for i in range(zpt // 2048):
        pltpu.sync_copy(zerov, tbl.at[pl.ds(s * zpt + i * 2048, 2048)])
    plsc.subcore_barrier()

    lane = lax.iota(jnp.int32, 16)
    half_lo = c * HALF

    def body(k, carry):
        ci = s + k * NS
        base = ci * CHUNK
        ca = pltpu.async_copy(dst_hbm.at[pl.ds(base, CHUNK)], dstv, sem)
        cb = pltpu.async_copy(ty_hbm.at[pl.ds(base, CHUNK)], tyv, sem2)
        ca.wait()
        cb.wait()
        for j in range(CHUNK // 16):
            sl = pl.ds(j * 16, 16)
            idx = dstv[sl] * R + tyv[sl]
            loc = idx - half_lo
            oor = (loc < 0) | (loc >= HALF)
            dump = HALF + ((base + j * 16 + lane) & DUMP_MASK)
            idxv[sl] = jnp.where(oor, dump, loc)
        pltpu.sync_copy(onesv, tbl.at[idxv], add=True)
        return carry

    nck = 390 + (s < 10).astype(jnp.int32)
    lax.fori_loop(0, nck, body, 0)
    plsc.subcore_barrier()
    wpt = HALF // NS
    for i in range(wpt // 2048):
        off = s * wpt + i * 2048
        pltpu.sync_copy(tbl.at[pl.ds(off, 2048)], zerov)
        pltpu.sync_copy(zerov, deg_hbm.at[pl.ds(c * HALF + off, 2048)])


def _gather_body(with_w):
    def body(*args):
        if with_w:
            (src_hbm, dst_hbm, ty_hbm, z_hbm, deg_hbm,
             ylo_hbm, yhi_hbm, w_hbm,
             srcv, dstv, tyv, zidxv, widxv, degv, wv, rows, rlo, rhi,
             sem, sem2, sem3) = args
        else:
            (src_hbm, ty_hbm, z_hbm, win_hbm,
             ylo_hbm, yhi_hbm,
             srcv, tyv, zidxv, wv, rows, rlo, rhi, sem, sem2, sem3) = args
        c = lax.axis_index("c")
        s = lax.axis_index("s")
        wid = s * NC + c

        def scale(j, carry):
            jv = jnp.full((16,), j, jnp.int32)
            wj = plsc.load_gather(wv, [jv])
            for t in range(WLO // 16):
                sl = pl.ds(t * 16, 16)
                rlo[j, sl] = rows[j, sl] * wj
            for t in range(WHI // 16):
                rhi[j, pl.ds(t * 16, 16)] = rows[j, pl.ds(WLO + t * 16, 16)] * wj
            return carry

        def step(k, carry):
            base = (wid + k * NT) * CHUNK
            ca = pltpu.async_copy(src_hbm.at[pl.ds(base, CHUNK)], srcv, sem)
            cb = pltpu.async_copy(ty_hbm.at[pl.ds(base, CHUNK)], tyv, sem2)
            if with_w:
                cc = pltpu.async_copy(dst_hbm.at[pl.ds(base, CHUNK)], dstv,
                                      sem3)
            else:
                cc = pltpu.async_copy(win_hbm.at[pl.ds(base, CHUNK)], wv, sem3)
            ca.wait()
            cb.wait()
            cc.wait()
            for j in range(CHUNK // 16):
                sl = pl.ds(j * 16, 16)
                zidxv[sl] = tyv[sl] * N + srcv[sl]
            if with_w:
                for j in range(CHUNK // 16):
                    sl = pl.ds(j * 16, 16)
                    widxv[sl] = dstv[sl] * R + tyv[sl]
                cz = pltpu.async_copy(z_hbm.at[zidxv], rows, sem)
                pltpu.async_copy(deg_hbm.at[widxv], degv, sem2).wait()
                for j in range(CHUNK // 16):
                    sl = pl.ds(j * 16, 16)
                    wv[sl] = 1.0 / degv[sl]
                cw = pltpu.async_copy(wv, w_hbm.at[pl.ds(base, CHUNK)], sem3)
                cz.wait()
            else:
                pltpu.async_copy(z_hbm.at[zidxv], rows, sem).wait()
            lax.fori_loop(0, CHUNK, scale, 0)
            wa = pltpu.async_copy(rlo, ylo_hbm.at[pl.ds(base, CHUNK)], sem)
            wb = pltpu.async_copy(rhi, yhi_hbm.at[pl.ds(base, CHUNK)], sem2)
            if with_w:
                cw.wait()
            wa.wait()
            wb.wait()
            return carry

        nck = 195 + (wid < 10).astype(jnp.int32)
        lax.fori_loop(0, nck, step, 0)

    return body


_gatherw_kernel = functools.partial(
    pl.kernel,
    mesh=_mesh,
    compiler_params=_sc_params,
    out_type=(jax.ShapeDtypeStruct((E, WLO), jnp.float32),
              jax.ShapeDtypeStruct((E, WHI), jnp.float32),
              jax.ShapeDtypeStruct((E,), jnp.float32)),
    scratch_types=[
        pltpu.VMEM((CHUNK,), jnp.int32),
        pltpu.VMEM((CHUNK,), jnp.int32),
        pltpu.VMEM((CHUNK,), jnp.int32),
        pltpu.VMEM((CHUNK,), jnp.int32),
        pltpu.VMEM((CHUNK,), jnp.int32),
        pltpu.VMEM((CHUNK,), jnp.float32),
        pltpu.VMEM((CHUNK,), jnp.float32),
        pltpu.VMEM((CHUNK, D), jnp.float32),
        pltpu.VMEM((CHUNK, WLO), jnp.float32),
        pltpu.VMEM((CHUNK, WHI), jnp.float32),
        pltpu.SemaphoreType.DMA,
        pltpu.SemaphoreType.DMA,
        pltpu.SemaphoreType.DMA,
    ],
)(_gather_body(True))

_gather_kernel = functools.partial(
    pl.kernel,
    mesh=_mesh,
    compiler_params=_sc_params,
    out_type=(jax.ShapeDtypeStruct((E, WLO), jnp.float32),
              jax.ShapeDtypeStruct((E, WHI), jnp.float32)),
    scratch_types=[
        pltpu.VMEM((CHUNK,), jnp.int32),
        pltpu.VMEM((CHUNK,), jnp.int32),
        pltpu.VMEM((CHUNK,), jnp.int32),
        pltpu.VMEM((CHUNK,), jnp.float32),
        pltpu.VMEM((CHUNK, D), jnp.float32),
        pltpu.VMEM((CHUNK, WLO), jnp.float32),
        pltpu.VMEM((CHUNK, WHI), jnp.float32),
        pltpu.SemaphoreType.DMA,
        pltpu.SemaphoreType.DMA,
        pltpu.SemaphoreType.DMA,
    ],
)(_gather_body(False))


def _ztable_body(x_ref, wd_ref, z_ref):
    z_ref[...] = jnp.dot(x_ref[...], wd_ref[0],
                         preferred_element_type=jnp.float32)


def _ztable(x, wd):
    nt = N // TNZ
    return pl.pallas_call(
        _ztable_body,
        grid=(nt, R),
        in_specs=[
            pl.BlockSpec((TNZ, D), lambda i, r: (i, 0)),
            pl.BlockSpec((1, D, D), lambda i, r: (r, 0, 0)),
        ],
        out_specs=pl.BlockSpec((TNZ, D), lambda i, r: (r * nt + i, 0)),
        out_shape=jax.ShapeDtypeStruct((R * N, D), jnp.float32),
    )(x, wd)


def _make_scatter(width):
    @functools.partial(
        pl.kernel,
        mesh=_mesh,
        compiler_params=_sc_params,
        out_type=jax.ShapeDtypeStruct((2 * ACC_ROWS, width), jnp.float32),
        scratch_types=[
            pltpu.VMEM((CHUNK,), jnp.int32),
            pltpu.VMEM((CHUNK,), jnp.int32),
            pltpu.VMEM((CHUNK, width), jnp.float32),
            pltpu.VMEM((CHUNK, width), jnp.float32),
            pltpu.VMEM_SHARED((ACC_ROWS, width), jnp.float32),
            pltpu.SemaphoreType.DMA,
            pltpu.SemaphoreType.DMA,
        ],
    )
    def _scatter_kernel(dst_hbm, y_hbm, out_hbm, dstv, idxv, rows, zrows, acc,
                        sem, sem2):
        c = lax.axis_index("c")
        s = lax.axis_index("s")
        for j in range(CHUNK):
            _fill(zrows.at[j], width, 0.0, jnp.float32)
        rpt = ACC_ROWS // NS
        for i in range(rpt // CHUNK):
            pltpu.sync_copy(zrows, acc.at[pl.ds(s * rpt + i * CHUNK, CHUNK)])
        pltpu.sync_copy(
            zrows.at[pl.ds(0, rpt % CHUNK)],
            acc.at[pl.ds(s * rpt + (rpt // CHUNK) * CHUNK, rpt % CHUNK)])
        plsc.subcore_barrier()

        lane = lax.iota(jnp.int32, 16)
        base_node = c * NHALF

        def step(k, carry):
            base = (s + k * NS) * CHUNK
            ca = pltpu.async_copy(dst_hbm.at[pl.ds(base, CHUNK)], dstv, sem)
            cb = pltpu.async_copy(y_hbm.at[pl.ds(base, CHUNK)], rows, sem2)
            ca.wait()
            cb.wait()
            for j in range(CHUNK // 16):
                sl = pl.ds(j * 16, 16)
                loc = dstv[sl] - base_node
                oor = (loc < 0) | (loc >= NHALF)
                dump = SC_DUMP + ((j * 16 + lane) & 63)
                idxv[sl] = jnp.where(oor, dump, loc)
            pltpu.sync_copy(rows, acc.at[idxv], add=True)
            return carry

        nck = 390 + (s < 10).astype(jnp.int32)
        lax.fori_loop(0, nck, step, 0)
        plsc.subcore_barrier()
        for i in range(rpt // CHUNK):
            off = s * rpt + i * CHUNK
            pltpu.sync_copy(acc.at[pl.ds(off, CHUNK)], rows)
            pltpu.sync_copy(rows, out_hbm.at[pl.ds(c * ACC_ROWS + off, CHUNK)])
        rem = rpt % CHUNK
        off = s * rpt + (rpt // CHUNK) * CHUNK
        pltpu.sync_copy(acc.at[pl.ds(off, rem)], rows.at[pl.ds(0, rem)])
        pltpu.sync_copy(rows.at[pl.ds(0, rem)],
                        out_hbm.at[pl.ds(c * ACC_ROWS + off, rem)])

    return _scatter_kernel


_scatter_lo = _make_scatter(WLO)
_scatter_hi = _make_scatter(WHI)


def _root_body(relu):
    def body(x_ref, olo_ref, ohi_ref, root_ref, b_ref, y_ref):
        o = jnp.concatenate([olo_ref[...], ohi_ref[...]], axis=1)
        y = o + jnp.dot(x_ref[...], root_ref[...],
                        preferred_element_type=jnp.float32) + b_ref[...]
        if relu:
            y = jnp.maximum(y, 0.0)
        y_ref[...] = y
    return body


def _root_call(x, olo, ohi, root, bias, relu):
    return pl.pallas_call(
        _root_body(relu),
        grid=(N // TN,),
        in_specs=[
            pl.BlockSpec((TN, D), lambda i: (i, 0)),
            pl.BlockSpec((TN, WLO), lambda i: (i, 0)),
            pl.BlockSpec((TN, WHI), lambda i: (i, 0)),
            pl.BlockSpec((D, D), lambda i: (0, 0)),
            pl.BlockSpec((1, D), lambda i: (0, 0)),
        ],
        out_specs=pl.BlockSpec((TN, D), lambda i: (i, 0)),
        out_shape=jax.ShapeDtypeStruct((N, D), jnp.float32),
    )(x, olo, ohi, root, bias)


def _block_diag_dense(w):
    out = jnp.zeros((R, NB, BLK, NB, BLK), w.dtype)
    for b in range(NB):
        out = out.at[:, b, :, b, :].set(w[:, b])
    return out.reshape(R, D, D)


def _unpad(outh):
    return jnp.concatenate(
        [outh[:NHALF], outh[ACC_ROWS:ACC_ROWS + NHALF]], axis=0)


def kernel(node_emb, weight1, root1, bias1, weight2, root2, bias2,
           edge_index, edge_type):
    src = edge_index[0].astype(jnp.int32)
    dst = edge_index[1].astype(jnp.int32)
    ty = edge_type.astype(jnp.int32)
    wd1 = _block_diag_dense(weight1)
    wd2 = _block_diag_dense(weight2)
    b1 = bias1.reshape(1, D)
    b2 = bias2.reshape(1, D)

    deg = _deg_kernel(dst, ty)

    z1 = _ztable(node_emb, wd1)
    y1lo, y1hi, w = _gatherw_kernel(src, dst, ty, z1, deg)
    o1lo = _unpad(_scatter_lo(dst, y1lo))
    o1hi = _unpad(_scatter_hi(dst, y1hi))
    x1 = _root_call(node_emb, o1lo, o1hi, root1, b1, relu=True)

    z2 = _ztable(x1, wd2)
    y2lo, y2hi = _gather_kernel(src, ty, z2, w)
    o2lo = _unpad(_scatter_lo(dst, y2lo))
    o2hi = _unpad(_scatter_hi(dst, y2hi))
    x2 = _root_call(x1, o2lo, o2hi, root2, b2, relu=False)
    return x2
